# baseline (speedup 1.0000x reference)
import jax
import jax.numpy as jnp
from jax.experimental import pallas as pl
from jax.experimental.pallas import tpu as pltpu

B, T, CIN, N = 4, 16, 68, 128
HD = 128
K = 16
NG = 4 * HD

_MM = (((1,), (0,)), ((), ()))


def _step_kernel(x_ref, xp_ref, post_ref, wxb_ref, woh_ref, out_ref,
                 h_ref, c_ref):
    t = pl.program_id(0)
    f32 = jnp.float32

    @pl.when(t == 0)
    def _init():
        h_ref[...] = jnp.zeros_like(h_ref)
        c_ref[...] = jnp.zeros_like(c_ref)

    iota_m = jax.lax.broadcasted_iota(jnp.int32, (N, N), 0)
    BIG = f32(3.0e38)

    for b in range(B):
        xt = x_ref[0, b]
        pos_t = xt[:4, :]
        pos_prev = xp_ref[0, b]
        ppt = post_ref[0, b]
        h_prev = h_ref[b]
        c_prev = c_ref[b]

        A = jax.lax.dot_general(wxb_ref[...], xt, _MM,
                                preferred_element_type=f32)
        ph = jnp.concatenate([pos_prev, h_prev], axis=0)
        Bm = jax.lax.dot_general(woh_ref[...], ph, _MM,
                                 preferred_element_type=f32)

        E = jnp.zeros((N, N), f32)
        for ci in range(4):
            d = ppt[:, ci:ci + 1] - pos_t[ci:ci + 1, :]
            E = E + d * d

        hmax = jnp.full((HD, N), -BIG, f32)
        cmax = jnp.full((HD, N), -BIG, f32)
        for _ in range(K):
            v = jnp.min(E, axis=0, keepdims=True)
            eq = E == v
            am = jnp.min(jnp.where(eq, iota_m, N), axis=0,
                         keepdims=True)
            sel = iota_m == am
            E = jnp.where(sel, BIG, E)
            onehot = sel.astype(f32)
            Gg = jax.lax.dot_general(Bm, onehot, _MM,
                                     preferred_element_type=f32)
            cg = jax.lax.dot_general(c_prev, onehot, _MM,
                                     preferred_element_type=f32)
            g = A + Gg
            th_i = jnp.tanh(g[0:HD])
            th_f = jnp.tanh(g[HD:2 * HD])
            th_o = jnp.tanh(g[2 * HD:3 * HD])
            t_g = jnp.tanh(g[3 * HD:4 * HD])
            cn = 0.5 * ((th_f + 1.0) * cg + (th_i + 1.0) * t_g)
            hn = (0.5 * th_o + 0.5) * jnp.tanh(cn)
            hmax = jnp.maximum(hmax, hn)
            cmax = jnp.maximum(cmax, cn)

        h_ref[b] = hmax
        c_ref[b] = cmax
        out_ref[0, b] = hmax


def _run(xs1, pos, post, wxb, woh):
    prev = lambda t: (jnp.maximum(t - 1, 0), 0, 0, 0)
    return pl.pallas_call(
        _step_kernel,
        grid=(T,),
        in_specs=[
            pl.BlockSpec((1, B, CIN + 1, N), lambda t: (t, 0, 0, 0)),
            pl.BlockSpec((1, B, 4, N), prev),
            pl.BlockSpec((1, B, N, 4), prev),
            pl.BlockSpec((NG, CIN + 1), lambda t: (0, 0)),
            pl.BlockSpec((NG, 4 + HD), lambda t: (0, 0)),
        ],
        out_specs=pl.BlockSpec((1, B, HD, N), lambda t: (t, 0, 0, 0)),
        out_shape=jax.ShapeDtypeStruct((T, B, HD, N), jnp.float32),
        scratch_shapes=[
            pltpu.VMEM((B, HD, N), jnp.float32),
            pltpu.VMEM((B, HD, N), jnp.float32),
        ],
        compiler_params=pltpu.CompilerParams(
            dimension_semantics=("arbitrary",)),
    )(xs1, pos, post, wxb, woh)


def kernel(input_tensor, W, b):
    x = input_tensor
    xs = x.transpose(1, 0, 2, 3)
    pos = xs[:, :, :4, :]
    post = pos.transpose(0, 1, 3, 2)
    ones = jnp.ones((T, B, 1, N), jnp.float32)
    xs1 = jnp.concatenate([xs, ones], axis=2)
    wx = W[:, :CIN].at[:, :4].add(-W[:, CIN:CIN + 4])
    scale = jnp.concatenate([jnp.full((3 * HD, 1), 0.5, jnp.float32),
                             jnp.ones((HD, 1), jnp.float32)], axis=0)
    wxb = jnp.concatenate([wx, b[:, None]], axis=1) * scale
    woh = W[:, CIN:] * scale
    h_out = _run(xs1, pos, post, wxb, woh)
    return jnp.concatenate([x[:, :, :4, :], h_out.transpose(1, 0, 2, 3)],
                           axis=2)


# speedup vs baseline: 1.0319x; 1.0319x over previous
import jax
import jax.numpy as jnp
from jax.experimental import pallas as pl
from jax.experimental.pallas import tpu as pltpu

B, T, CIN, N = 4, 16, 68, 128
HD = 128
K = 16
NG = 4 * HD

_MM = (((1,), (0,)), ((), ()))


def _lstm_kernel(x_ref, post_ref, wxb_ref, woh_ref, out_ref, h_ref, c_ref):
    f32 = jnp.float32
    iota_m = jax.lax.broadcasted_iota(jnp.int32, (N, N), 0)
    BIG = f32(3.0e38)

    h_ref[...] = jnp.zeros_like(h_ref)
    c_ref[...] = jnp.zeros_like(c_ref)

    def step(t, carry):
        tp = jnp.maximum(t - 1, 0)
        for b in range(B):
            xt = x_ref[t, b]
            pos_t = xt[:4, :]
            pos_prev = x_ref[tp, b, :4, :]
            ppt = post_ref[tp, b]
            h_prev = h_ref[b]
            c_prev = c_ref[b]

            A = jax.lax.dot_general(wxb_ref[...], xt, _MM,
                                    preferred_element_type=f32)
            ph = jnp.concatenate([pos_prev, h_prev], axis=0)
            Bm = jax.lax.dot_general(woh_ref[...], ph, _MM,
                                     preferred_element_type=f32)

            E = jnp.zeros((N, N), f32)
            for ci in range(4):
                d = ppt[:, ci:ci + 1] - pos_t[ci:ci + 1, :]
                E = E + d * d

            hmax = jnp.full((HD, N), -BIG, f32)
            cmax = jnp.full((HD, N), -BIG, f32)
            for _ in range(K):
                v = jnp.min(E, axis=0, keepdims=True)
                eq = E == v
                am = jnp.min(jnp.where(eq, iota_m, N), axis=0,
                             keepdims=True)
                sel = iota_m == am
                E = jnp.where(sel, BIG, E)
                onehot = sel.astype(f32)
                Gg = jax.lax.dot_general(Bm, onehot, _MM,
                                         preferred_element_type=f32)
                cg = jax.lax.dot_general(c_prev, onehot, _MM,
                                         preferred_element_type=f32)
                g = A + Gg
                th_i = jnp.tanh(g[0:HD])
                th_f = jnp.tanh(g[HD:2 * HD])
                th_o = jnp.tanh(g[2 * HD:3 * HD])
                t_g = jnp.tanh(g[3 * HD:4 * HD])
                cn = 0.5 * ((th_f + 1.0) * cg + (th_i + 1.0) * t_g)
                hn = (0.5 * th_o + 0.5) * jnp.tanh(cn)
                hmax = jnp.maximum(hmax, hn)
                cmax = jnp.maximum(cmax, cn)

            h_ref[b] = hmax
            c_ref[b] = cmax
            out_ref[t, b] = hmax
        return carry

    jax.lax.fori_loop(0, T, step, 0)


def _run(xs1, post, wxb, woh):
    return pl.pallas_call(
        _lstm_kernel,
        in_specs=[
            pl.BlockSpec((T, B, CIN + 1, N), lambda: (0, 0, 0, 0)),
            pl.BlockSpec((T, B, N, 4), lambda: (0, 0, 0, 0)),
            pl.BlockSpec((NG, CIN + 1), lambda: (0, 0)),
            pl.BlockSpec((NG, 4 + HD), lambda: (0, 0)),
        ],
        out_specs=pl.BlockSpec((T, B, HD, N), lambda: (0, 0, 0, 0)),
        out_shape=jax.ShapeDtypeStruct((T, B, HD, N), jnp.float32),
        scratch_shapes=[
            pltpu.VMEM((B, HD, N), jnp.float32),
            pltpu.VMEM((B, HD, N), jnp.float32),
        ],
    )(xs1, post, wxb, woh)


def kernel(input_tensor, W, b):
    x = input_tensor
    xs = x.transpose(1, 0, 2, 3)
    post = xs[:, :, :4, :].transpose(0, 1, 3, 2)
    ones = jnp.ones((T, B, 1, N), jnp.float32)
    xs1 = jnp.concatenate([xs, ones], axis=2)
    wx = W[:, :CIN].at[:, :4].add(-W[:, CIN:CIN + 4])
    scale = jnp.concatenate([jnp.full((3 * HD, 1), 0.5, jnp.float32),
                             jnp.ones((HD, 1), jnp.float32)], axis=0)
    wxb = jnp.concatenate([wx, b[:, None]], axis=1) * scale
    woh = W[:, CIN:] * scale
    h_out = _run(xs1, post, wxb, woh)
    return jnp.concatenate([x[:, :, :4, :], h_out.transpose(1, 0, 2, 3)],
                           axis=2)


# speedup vs baseline: 1.0897x; 1.0559x over previous
import jax
import jax.numpy as jnp
from jax.experimental import pallas as pl
from jax.experimental.pallas import tpu as pltpu

B, T, CIN, N = 4, 16, 68, 128
HD = 128
K = 16
NG = 4 * HD

_MM = (((1,), (0,)), ((), ()))


def _lstm_kernel(x_ref, post_ref, wx_ref, wohb_ref, out_ref, h_ref, c_ref):
    f32 = jnp.float32
    iota_m = jax.lax.broadcasted_iota(jnp.int32, (N, N), 0)
    BIG = f32(3.0e38)
    ones_row = jnp.ones((1, N), f32)

    h_ref[...] = jnp.zeros_like(h_ref)
    c_ref[...] = jnp.zeros_like(c_ref)

    def step(t, carry):
        tp = jnp.maximum(t - 1, 0)
        for b in range(B):
            xt = x_ref[b, t]
            pos_t = xt[:4, :]
            pos_prev = x_ref[b, tp, :4, :]
            ppt = post_ref[b, tp]
            h_prev = h_ref[b]
            c_prev = c_ref[b]

            A = jax.lax.dot_general(wx_ref[...], xt, _MM,
                                    preferred_element_type=f32)
            ph = jnp.concatenate([pos_prev, h_prev, ones_row],
                                 axis=0)
            Bm = jax.lax.dot_general(wohb_ref[...], ph, _MM,
                                     preferred_element_type=f32)

            E = jnp.zeros((N, N), f32)
            for ci in range(4):
                d = ppt[:, ci:ci + 1] - pos_t[ci:ci + 1, :]
                E = E + d * d

            hmax = jnp.full((HD, N), -BIG, f32)
            cmax = jnp.full((HD, N), -BIG, f32)
            for _ in range(K):
                v = jnp.min(E, axis=0, keepdims=True)
                eq = E == v
                am = jnp.min(jnp.where(eq, iota_m, N), axis=0,
                             keepdims=True)
                sel = iota_m == am
                E = jnp.where(sel, BIG, E)
                onehot = sel.astype(f32)
                Gg = jax.lax.dot_general(Bm, onehot, _MM,
                                         preferred_element_type=f32)
                cg = jax.lax.dot_general(c_prev, onehot, _MM,
                                         preferred_element_type=f32)
                g = A + Gg
                th_i = jnp.tanh(g[0:HD])
                th_f = jnp.tanh(g[HD:2 * HD])
                th_o = jnp.tanh(g[2 * HD:3 * HD])
                t_g = jnp.tanh(g[3 * HD:4 * HD])
                cn = 0.5 * ((th_f + 1.0) * cg + (th_i + 1.0) * t_g)
                hn = (0.5 * th_o + 0.5) * jnp.tanh(cn)
                hmax = jnp.maximum(hmax, hn)
                cmax = jnp.maximum(cmax, cn)

            h_ref[b] = hmax
            c_ref[b] = cmax
            out_ref[b, t, :4, :] = pos_t
            out_ref[b, t, 4:, :] = hmax
        return carry

    jax.lax.fori_loop(0, T, step, 0)


def _run(x, post, wx, wohb):
    return pl.pallas_call(
        _lstm_kernel,
        in_specs=[
            pl.BlockSpec((B, T, CIN, N), lambda: (0, 0, 0, 0)),
            pl.BlockSpec((B, T, N, 4), lambda: (0, 0, 0, 0)),
            pl.BlockSpec((NG, CIN), lambda: (0, 0)),
            pl.BlockSpec((NG, 4 + HD + 1), lambda: (0, 0)),
        ],
        out_specs=pl.BlockSpec((B, T, 4 + HD, N), lambda: (0, 0, 0, 0)),
        out_shape=jax.ShapeDtypeStruct((B, T, 4 + HD, N), jnp.float32),
        scratch_shapes=[
            pltpu.VMEM((B, HD, N), jnp.float32),
            pltpu.VMEM((B, HD, N), jnp.float32),
        ],
    )(x, post, wx, wohb)


def kernel(input_tensor, W, b):
    x = input_tensor
    post = x[:, :, :4, :].transpose(0, 1, 3, 2)
    scale = jnp.concatenate([jnp.full((3 * HD, 1), 0.5, jnp.float32),
                             jnp.ones((HD, 1), jnp.float32)], axis=0)
    wx = (W[:, :CIN].at[:, :4].add(-W[:, CIN:CIN + 4])) * scale
    wohb = jnp.concatenate([W[:, CIN:], b[:, None]], axis=1) * scale
    return _run(x, post, wx, wohb)


# speedup vs baseline: 1.1689x; 1.0728x over previous
import jax
import jax.numpy as jnp
from jax.experimental import pallas as pl
from jax.experimental.pallas import tpu as pltpu

B, T, CIN, N = 4, 16, 68, 128
HD = 128
K = 16
NG = 4 * HD

_MM = (((1,), (0,)), ((), ()))


def _lstm_kernel(x_ref, post_ref, wx_ref, wohb_ref, out_ref, h_ref, c_ref):
    f32 = jnp.float32
    iota_m = jax.lax.broadcasted_iota(jnp.int32, (N, N), 0)
    BIG = f32(3.0e38)
    ones_row = jnp.ones((1, N), f32)

    h_ref[...] = jnp.zeros_like(h_ref)
    c_ref[...] = jnp.zeros_like(c_ref)

    def step(t, carry):
        tp = jnp.maximum(t - 1, 0)
        for b in range(B):
            xt = x_ref[b, t]
            pos_t = xt[:4, :]
            pos_prev = x_ref[b, tp, :4, :]
            ppt = post_ref[b, tp]
            h_prev = h_ref[b]
            c_prev = c_ref[b]

            A = jax.lax.dot_general(wx_ref[...], xt, _MM,
                                    preferred_element_type=f32)
            ph = jnp.concatenate([pos_prev, h_prev, ones_row],
                                 axis=0)
            Bm = jax.lax.dot_general(wohb_ref[...], ph, _MM,
                                     preferred_element_type=f32)

            E = jnp.zeros((N, N), f32)
            for ci in range(4):
                d = ppt[:, ci:ci + 1] - pos_t[ci:ci + 1, :]
                E = E + d * d

            hmax = jnp.full((HD, N), -BIG, f32)
            cmax = jnp.full((HD, N), -BIG, f32)
            for _ in range(K):
                am = jax.lax.argmin(E, 0, jnp.int32)[None, :]
                sel = iota_m == am
                E = jnp.where(sel, BIG, E)
                onehot = sel.astype(f32)
                Gg = jax.lax.dot_general(Bm, onehot, _MM,
                                         preferred_element_type=f32)
                cg = jax.lax.dot_general(c_prev, onehot, _MM,
                                         preferred_element_type=f32)
                g = A + Gg
                th_i = jnp.tanh(g[0:HD])
                th_f = jnp.tanh(g[HD:2 * HD])
                th_o = jnp.tanh(g[2 * HD:3 * HD])
                t_g = jnp.tanh(g[3 * HD:4 * HD])
                cn = 0.5 * ((th_f + 1.0) * cg + (th_i + 1.0) * t_g)
                hn = (0.5 * th_o + 0.5) * jnp.tanh(cn)
                hmax = jnp.maximum(hmax, hn)
                cmax = jnp.maximum(cmax, cn)

            h_ref[b] = hmax
            c_ref[b] = cmax
            out_ref[b, t, :4, :] = pos_t
            out_ref[b, t, 4:, :] = hmax
        return carry

    jax.lax.fori_loop(0, T, step, 0)


def _run(x, post, wx, wohb):
    return pl.pallas_call(
        _lstm_kernel,
        in_specs=[
            pl.BlockSpec((B, T, CIN, N), lambda: (0, 0, 0, 0)),
            pl.BlockSpec((B, T, N, 4), lambda: (0, 0, 0, 0)),
            pl.BlockSpec((NG, CIN), lambda: (0, 0)),
            pl.BlockSpec((NG, 4 + HD + 1), lambda: (0, 0)),
        ],
        out_specs=pl.BlockSpec((B, T, 4 + HD, N), lambda: (0, 0, 0, 0)),
        out_shape=jax.ShapeDtypeStruct((B, T, 4 + HD, N), jnp.float32),
        scratch_shapes=[
            pltpu.VMEM((B, HD, N), jnp.float32),
            pltpu.VMEM((B, HD, N), jnp.float32),
        ],
    )(x, post, wx, wohb)


def kernel(input_tensor, W, b):
    x = input_tensor
    post = x[:, :, :4, :].transpose(0, 1, 3, 2)
    scale = jnp.concatenate([jnp.full((3 * HD, 1), 0.5, jnp.float32),
                             jnp.ones((HD, 1), jnp.float32)], axis=0)
    wx = (W[:, :CIN].at[:, :4].add(-W[:, CIN:CIN + 4])) * scale
    wohb = jnp.concatenate([W[:, CIN:], b[:, None]], axis=1) * scale
    return _run(x, post, wx, wohb)


# speedup vs baseline: 1.1860x; 1.0146x over previous
import jax
import jax.numpy as jnp
from jax.experimental import pallas as pl
from jax.experimental.pallas import tpu as pltpu

B, T, CIN, N = 4, 16, 68, 128
HD = 128
K = 16
NG = 4 * HD

_MM = (((1,), (0,)), ((), ()))


def _lstm_kernel(x_ref, post_ref, wx_ref, wohb_ref, out_ref, h_ref, c_ref):
    f32 = jnp.float32
    iota_m = jax.lax.broadcasted_iota(jnp.int32, (N, N), 0)
    BIG = f32(3.0e38)
    ones_row = jnp.ones((1, N), f32)

    h_ref[...] = jnp.zeros_like(h_ref)
    c_ref[...] = jnp.zeros_like(c_ref)

    def step(i, carry):
      for dt in range(2):
        t = 2 * i + dt
        tp = jnp.maximum(t - 1, 0)
        for b in range(B):
            xt = x_ref[b, t]
            pos_t = xt[:4, :]
            pos_prev = x_ref[b, tp, :4, :]
            ppt = post_ref[b, tp]
            h_prev = h_ref[b]
            c_prev = c_ref[b]

            A = jax.lax.dot_general(wx_ref[...], xt, _MM,
                                    preferred_element_type=f32)
            ph = jnp.concatenate([pos_prev, h_prev, ones_row],
                                 axis=0)
            Bm = jax.lax.dot_general(wohb_ref[...], ph, _MM,
                                     preferred_element_type=f32)

            E = jnp.zeros((N, N), f32)
            for ci in range(4):
                d = ppt[:, ci:ci + 1] - pos_t[ci:ci + 1, :]
                E = E + d * d

            hmax = jnp.full((HD, N), -BIG, f32)
            cmax = jnp.full((HD, N), -BIG, f32)
            for _ in range(K):
                am = jax.lax.argmin(E, 0, jnp.int32)[None, :]
                sel = iota_m == am
                E = jnp.where(sel, BIG, E)
                onehot = sel.astype(f32)
                Gg = jax.lax.dot_general(Bm, onehot, _MM,
                                         preferred_element_type=f32)
                cg = jax.lax.dot_general(c_prev, onehot, _MM,
                                         preferred_element_type=f32)
                g = A + Gg
                th_i = jnp.tanh(g[0:HD])
                th_f = jnp.tanh(g[HD:2 * HD])
                th_o = jnp.tanh(g[2 * HD:3 * HD])
                t_g = jnp.tanh(g[3 * HD:4 * HD])
                cn = 0.5 * ((th_f + 1.0) * cg + (th_i + 1.0) * t_g)
                hn = (0.5 * th_o + 0.5) * jnp.tanh(cn)
                hmax = jnp.maximum(hmax, hn)
                cmax = jnp.maximum(cmax, cn)

            h_ref[b] = hmax
            c_ref[b] = cmax
            out_ref[b, t, :4, :] = pos_t
            out_ref[b, t, 4:, :] = hmax
      return carry

    jax.lax.fori_loop(0, T // 2, step, 0)


def _run(x, post, wx, wohb):
    return pl.pallas_call(
        _lstm_kernel,
        in_specs=[
            pl.BlockSpec((B, T, CIN, N), lambda: (0, 0, 0, 0)),
            pl.BlockSpec((B, T, N, 4), lambda: (0, 0, 0, 0)),
            pl.BlockSpec((NG, CIN), lambda: (0, 0)),
            pl.BlockSpec((NG, 4 + HD + 1), lambda: (0, 0)),
        ],
        out_specs=pl.BlockSpec((B, T, 4 + HD, N), lambda: (0, 0, 0, 0)),
        out_shape=jax.ShapeDtypeStruct((B, T, 4 + HD, N), jnp.float32),
        scratch_shapes=[
            pltpu.VMEM((B, HD, N), jnp.float32),
            pltpu.VMEM((B, HD, N), jnp.float32),
        ],
    )(x, post, wx, wohb)


def kernel(input_tensor, W, b):
    x = input_tensor
    post = x[:, :, :4, :].transpose(0, 1, 3, 2)
    scale = jnp.concatenate([jnp.full((3 * HD, 1), 0.5, jnp.float32),
                             jnp.ones((HD, 1), jnp.float32)], axis=0)
    wx = (W[:, :CIN].at[:, :4].add(-W[:, CIN:CIN + 4])) * scale
    wohb = jnp.concatenate([W[:, CIN:], b[:, None]], axis=1) * scale
    return _run(x, post, wx, wohb)


# speedup vs baseline: 1.2018x; 1.0133x over previous
import jax
import jax.numpy as jnp
from jax.experimental import pallas as pl
from jax.experimental.pallas import tpu as pltpu

B, T, CIN, N = 4, 16, 68, 128
HD = 128
K = 16
NG = 4 * HD

_MM = (((1,), (0,)), ((), ()))


def _lstm_kernel(x_ref, post_ref, wx_ref, wohb_ref, out_ref, h_ref, c_ref):
    f32 = jnp.float32
    iota_m = jax.lax.broadcasted_iota(jnp.int32, (N, N), 0)
    BIG = f32(3.0e38)
    ones_row = jnp.ones((1, N), f32)

    h_ref[...] = jnp.zeros_like(h_ref)
    c_ref[...] = jnp.zeros_like(c_ref)

    def step(i, carry):
      for dt in range(4):
        t = 4 * i + dt
        tp = jnp.maximum(t - 1, 0)
        for b in range(B):
            xt = x_ref[b, t]
            pos_t = xt[:4, :]
            pos_prev = x_ref[b, tp, :4, :]
            ppt = post_ref[b, tp]
            h_prev = h_ref[b]
            c_prev = c_ref[b]

            A = jax.lax.dot_general(wx_ref[...], xt, _MM,
                                    preferred_element_type=f32)
            ph = jnp.concatenate([pos_prev, h_prev, ones_row],
                                 axis=0)
            Bm = jax.lax.dot_general(wohb_ref[...], ph, _MM,
                                     preferred_element_type=f32)

            E = jnp.zeros((N, N), f32)
            for ci in range(4):
                d = ppt[:, ci:ci + 1] - pos_t[ci:ci + 1, :]
                E = E + d * d

            hmax = jnp.full((HD, N), -BIG, f32)
            cmax = jnp.full((HD, N), -BIG, f32)
            for _ in range(K):
                am = jax.lax.argmin(E, 0, jnp.int32)[None, :]
                sel = iota_m == am
                E = jnp.where(sel, BIG, E)
                onehot = sel.astype(f32)
                Gg = jax.lax.dot_general(Bm, onehot, _MM,
                                         preferred_element_type=f32)
                cg = jax.lax.dot_general(c_prev, onehot, _MM,
                                         preferred_element_type=f32)
                g = A + Gg
                th_i = jnp.tanh(g[0:HD])
                th_f = jnp.tanh(g[HD:2 * HD])
                th_o = jnp.tanh(g[2 * HD:3 * HD])
                t_g = jnp.tanh(g[3 * HD:4 * HD])
                cn = 0.5 * ((th_f + 1.0) * cg + (th_i + 1.0) * t_g)
                hn = (0.5 * th_o + 0.5) * jnp.tanh(cn)
                hmax = jnp.maximum(hmax, hn)
                cmax = jnp.maximum(cmax, cn)

            h_ref[b] = hmax
            c_ref[b] = cmax
            out_ref[b, t, :4, :] = pos_t
            out_ref[b, t, 4:, :] = hmax
      return carry

    jax.lax.fori_loop(0, T // 4, step, 0)


def _run(x, post, wx, wohb):
    return pl.pallas_call(
        _lstm_kernel,
        in_specs=[
            pl.BlockSpec((B, T, CIN, N), lambda: (0, 0, 0, 0)),
            pl.BlockSpec((B, T, N, 4), lambda: (0, 0, 0, 0)),
            pl.BlockSpec((NG, CIN), lambda: (0, 0)),
            pl.BlockSpec((NG, 4 + HD + 1), lambda: (0, 0)),
        ],
        out_specs=pl.BlockSpec((B, T, 4 + HD, N), lambda: (0, 0, 0, 0)),
        out_shape=jax.ShapeDtypeStruct((B, T, 4 + HD, N), jnp.float32),
        scratch_shapes=[
            pltpu.VMEM((B, HD, N), jnp.float32),
            pltpu.VMEM((B, HD, N), jnp.float32),
        ],
    )(x, post, wx, wohb)


def kernel(input_tensor, W, b):
    x = input_tensor
    post = x[:, :, :4, :].transpose(0, 1, 3, 2)
    scale = jnp.concatenate([jnp.full((3 * HD, 1), 0.5, jnp.float32),
                             jnp.ones((HD, 1), jnp.float32)], axis=0)
    wx = (W[:, :CIN].at[:, :4].add(-W[:, CIN:CIN + 4])) * scale
    wohb = jnp.concatenate([W[:, CIN:], b[:, None]], axis=1) * scale
    return _run(x, post, wx, wohb)
